# baseline (speedup 1.0000x reference)
import functools

import jax
import jax.numpy as jnp
from jax import lax
from jax.experimental import pallas as pl
from jax.experimental.pallas import tpu as pltpu
from jax.experimental.pallas import tpu_sc as plsc

NC = 2
NS = 16
NW = NC * NS
L = 16


@jax.jit
def _pad_kernel(tT, tailp):
    D, V = tT.shape
    NB = V // 128
    Vp = NB * 128 + tailp.shape[0]
    Dp = 2 * D

    mesh = plsc.VectorSubcoreMesh(core_axis_name="c", subcore_axis_name="s")

    @functools.partial(
        pl.kernel,
        mesh=mesh,
        out_type=jax.ShapeDtypeStruct((Vp, Dp), jnp.float32),
        scratch_types=[
            pltpu.VMEM((2, D, 128), jnp.float32),
            pltpu.VMEM((2, 128, Dp), jnp.float32),
            pltpu.SemaphoreType.DMA,
            pltpu.SemaphoreType.DMA,
            pltpu.SemaphoreType.DMA,
            pltpu.SemaphoreType.DMA,
        ],
        compiler_params=pltpu.CompilerParams(
            use_tc_tiling_on_sc=True, needs_layout_passes=False),
    )
    def k(tT_hbm, tailp_hbm, out_hbm, inb, outb, i0, i1, o0, o1):
        wid = lax.axis_index("s") * NC + lax.axis_index("c")
        isem = (i0, i1)
        osem = (o0, o1)
        n_g = (NB + NW - 1) // NW

        @pl.when(wid == 0)
        def _():
            pltpu.sync_copy(tailp_hbm, out_hbm.at[pl.ds(NB * 128, Vp - NB * 128)])

        def blk_of(g):
            return g * NW + wid

        def load(g, s):
            return pltpu.async_copy(
                tT_hbm.at[:, pl.ds(blk_of(g) * 128, 128)], inb.at[s], isem[s])

        def load_wait(g, s):
            pltpu.make_async_copy(
                tT_hbm.at[:, pl.ds(blk_of(g) * 128, 128)], inb.at[s], isem[s]
            ).wait()

        def store(g, s):
            return pltpu.async_copy(
                outb.at[s], out_hbm.at[pl.ds(blk_of(g) * 128, 128)], osem[s])

        def store_wait(g, s):
            pltpu.make_async_copy(
                outb.at[s], out_hbm.at[pl.ds(blk_of(g) * 128, 128)], osem[s]
            ).wait()

        lanes = jax.lax.broadcasted_iota(jnp.int32, (L,), 0)

        dg_lanes = [lanes + (dg * L) for dg in range(D // L)]
        zeros = jnp.zeros((L,), jnp.int32)

        def transpose(s):
            def row(i, carry):
                for dg in range(D // L):
                    v = plsc.load_gather(
                        inb.at[s], [dg_lanes[dg], zeros + i])
                    outb.at[s].at[i][pl.ds(dg * L, L)] = v
                return carry
            lax.fori_loop(0, 128, row, 0, unroll=8)

        @pl.when(blk_of(0) < NB)
        def _():
            load(0, 0)

        n_pairs = (n_g + 1) // 2

        def body(p, carry):
            for b in range(2):
                g = 2 * p + b
                s = b

                @pl.when((g >= 2) & (blk_of(g - 2) < NB))
                def _():
                    store_wait(g - 2, s)

                @pl.when(blk_of(g) < NB)
                def _():
                    load_wait(g, s)

                    @pl.when(blk_of(g + 1) < NB)
                    def _():
                        load(g + 1, 1 - s)
                    transpose(s)
                    store(g, s)
            return carry

        lax.fori_loop(0, n_pairs, body, 0)

        for gl in (2 * n_pairs - 2, 2 * n_pairs - 1):

            @pl.when(blk_of(gl) < NB)
            def _():
                store_wait(gl, gl % 2)

    return k(tT, tailp)


@functools.partial(jax.jit, static_argnames=("chunk",))
def _gather_rows(tpad, idx, chunk=400):
    N = idx.shape[0]
    Vp, Dp = tpad.shape
    per_w = N // NW
    n_chunks = per_w // chunk
    assert per_w % chunk == 0 and N % NW == 0 and n_chunks % 2 == 0

    mesh = plsc.VectorSubcoreMesh(core_axis_name="c", subcore_axis_name="s")

    @functools.partial(
        pl.kernel,
        mesh=mesh,
        out_type=jax.ShapeDtypeStruct((N, Dp), jnp.float32),
        scratch_types=[
            pltpu.VMEM((per_w,), jnp.int32),
            pltpu.VMEM((2, chunk, Dp), jnp.float32),
            pltpu.SemaphoreType.DMA,
            pltpu.SemaphoreType.DMA,
            pltpu.SemaphoreType.DMA,
            pltpu.SemaphoreType.DMA,
        ],
        compiler_params=pltpu.CompilerParams(use_tc_tiling_on_sc=True),
    )
    def k(tpad_hbm, idx_hbm, out_hbm, idx_v, rows_v, g0, g1, w0, w1):
        wid = lax.axis_index("s") * NC + lax.axis_index("c")
        base = wid * per_w
        gsem = (g0, g1)
        wsem = (w0, w1)

        pltpu.sync_copy(idx_hbm.at[pl.ds(base, per_w)], idx_v)

        def gather(i, slot):
            return pltpu.async_copy(
                tpad_hbm.at[idx_v.at[pl.ds(i * chunk, chunk)]],
                rows_v.at[slot], gsem[slot])

        def writeback(i, slot):
            return pltpu.async_copy(
                rows_v.at[slot],
                out_hbm.at[pl.ds(base + i * chunk, chunk)], wsem[slot])

        gather(0, 0)
        gather(1, 1)

        def body(p, carry):
            for b in range(2):
                i = 2 * p + b
                pltpu.make_async_copy(
                    tpad_hbm.at[idx_v.at[pl.ds(i * chunk, chunk)]],
                    rows_v.at[b], gsem[b]).wait()
                writeback(i, b)

                @pl.when(i + 2 < n_chunks)
                def _():
                    pltpu.make_async_copy(
                        rows_v.at[b],
                        out_hbm.at[pl.ds(base + i * chunk, chunk)],
                        wsem[b]).wait()
                    gather(i + 2, b)
            return carry

        lax.fori_loop(0, n_chunks // 2, body, 0)

        for b in range(2):
            i = n_chunks - 2 + b
            pltpu.make_async_copy(
                rows_v.at[b],
                out_hbm.at[pl.ds(base + i * chunk, chunk)],
                wsem[b]).wait()

    return k(tpad, idx)


def kernel(x, table):
    B, H = x.shape
    V, D = table.shape
    NB = V // 128
    tT = table.T
    tail_pad = (-(V - NB * 128)) % 8
    tailp = jnp.pad(table[NB * 128:], ((0, tail_pad), (0, D)))
    tpad = _pad_kernel(tT, tailp)
    flat = x.reshape(B * H)
    out = _gather_rows(tpad, flat)
    return out.reshape(B, H, 2 * D)[:, :, :D]


# speedup vs baseline: 1.9614x; 1.9614x over previous
import functools

import jax
import jax.numpy as jnp
from jax import lax
from jax.experimental import pallas as pl
from jax.experimental.pallas import tpu as pltpu
from jax.experimental.pallas import tpu_sc as plsc

NC = 2
NS = 16
NW = NC * NS


@functools.partial(jax.jit, static_argnames=("chunk",))
def _gather_rows(tpad, idx, chunk=400):
    N = idx.shape[0]
    Vp, Dp = tpad.shape
    D = Dp // 2
    per_w = N // NW
    n_chunks = per_w // chunk
    assert per_w % chunk == 0 and N % NW == 0 and n_chunks % 2 == 0

    mesh = plsc.VectorSubcoreMesh(core_axis_name="c", subcore_axis_name="s")

    @functools.partial(
        pl.kernel,
        mesh=mesh,
        out_type=jax.ShapeDtypeStruct((N, Dp), jnp.float32),
        scratch_types=[
            pltpu.VMEM((per_w,), jnp.int32),
            pltpu.VMEM((2, chunk, Dp), jnp.float32),
            pltpu.SemaphoreType.DMA,
            pltpu.SemaphoreType.DMA,
            pltpu.SemaphoreType.DMA,
            pltpu.SemaphoreType.DMA,
        ],
        compiler_params=pltpu.CompilerParams(use_tc_tiling_on_sc=True),
    )
    def k(tpad_hbm, idx_hbm, out_hbm, idx_v, rows_v, g0, g1, w0, w1):
        wid = lax.axis_index("s") * NC + lax.axis_index("c")
        base = wid * per_w
        gsem = (g0, g1)
        wsem = (w0, w1)

        pltpu.sync_copy(idx_hbm.at[pl.ds(base, per_w)], idx_v)

        def gather(i, slot):
            return pltpu.async_copy(
                tpad_hbm.at[idx_v.at[pl.ds(i * chunk, chunk)]],
                rows_v.at[slot], gsem[slot])

        def writeback(i, slot):
            return pltpu.async_copy(
                rows_v.at[slot],
                out_hbm.at[pl.ds(base + i * chunk, chunk)], wsem[slot])

        gather(0, 0)
        gather(1, 1)

        def body(p, carry):
            for b in range(2):
                i = 2 * p + b
                pltpu.make_async_copy(
                    tpad_hbm.at[idx_v.at[pl.ds(i * chunk, chunk)]],
                    rows_v.at[b], gsem[b]).wait()
                writeback(i, b)

                @pl.when(i + 2 < n_chunks)
                def _():
                    pltpu.make_async_copy(
                        rows_v.at[b],
                        out_hbm.at[pl.ds(base + i * chunk, chunk)],
                        wsem[b]).wait()
                    gather(i + 2, b)
            return carry

        lax.fori_loop(0, n_chunks // 2, body, 0)

        for b in range(2):
            i = n_chunks - 2 + b
            pltpu.make_async_copy(
                rows_v.at[b],
                out_hbm.at[pl.ds(base + i * chunk, chunk)],
                wsem[b]).wait()

    return k(tpad, idx)


def kernel(x, table):
    B, H = x.shape
    V, D = table.shape
    vpad = (-V) % 8
    tpad = jnp.pad(table, ((0, vpad), (0, D)))
    flat = x.reshape(B * H)
    out = _gather_rows(tpad, flat)
    return out.reshape(B, H, 2 * D)[:, :, :D]
